# baseline (speedup 1.0000x reference)
import functools

import jax
import jax.numpy as jnp
from jax import lax
from jax.experimental import pallas as pl
from jax.experimental.pallas import tpu as pltpu
from jax.experimental.pallas import tpu_sc as plsc

EMB = 64
SCALE = 8.0

NC = 2
NS = 16
NW = NC * NS
LANES = 16

B = 16384
L = 200
BT = B * L
PER_W = BT // NW
CHUNK = 1024
NSTREAM = CHUNK // 128
NITER = PER_W // CHUNK


def _gather_scale(idx2d, table):
    mesh = plsc.VectorSubcoreMesh(core_axis_name="c", subcore_axis_name="s")

    @functools.partial(
        pl.kernel,
        mesh=mesh,
        out_type=jax.ShapeDtypeStruct((BT, EMB), jnp.float32),
        scratch_types=[
            pltpu.VMEM((NSTREAM, 128), jnp.int32),
            pltpu.VMEM((CHUNK, EMB), jnp.float32),
            pltpu.SemaphoreType.DMA,
        ],
        compiler_params=pltpu.CompilerParams(use_tc_tiling_on_sc=False),
    )
    def k(idx_hbm, tab_hbm, out_hbm, idx_v, rows_v, sem):
        wid = lax.axis_index("s") * NC + lax.axis_index("c")
        base = wid * PER_W

        def body(i, carry):
            row0 = pl.multiple_of(base + i * CHUNK, CHUNK)
            irow0 = pl.multiple_of(row0 // 128, NSTREAM)
            pltpu.sync_copy(idx_hbm.at[pl.ds(irow0, NSTREAM)], idx_v)
            for j in range(NSTREAM):
                pltpu.async_copy(
                    tab_hbm.at[idx_v.at[j]],
                    rows_v.at[pl.ds(j * 128, 128)],
                    sem,
                )
            for j in range(NSTREAM):
                pltpu.make_async_copy(
                    tab_hbm.at[idx_v.at[j]],
                    rows_v.at[pl.ds(j * 128, 128)],
                    sem,
                ).wait()

            def scale_row(r, c2):
                for c in range(EMB // LANES):
                    sl = (r, pl.ds(c * LANES, LANES))
                    rows_v[sl] = rows_v[sl] * SCALE
                return c2

            lax.fori_loop(0, CHUNK, scale_row, 0)
            pltpu.sync_copy(rows_v, out_hbm.at[pl.ds(row0, CHUNK)])
            return carry

        lax.fori_loop(0, NITER, body, 0)

    return k(idx2d, table)


def kernel(token_tensor, embedding_table):
    idx = token_tensor.reshape(BT // 128, 128).astype(jnp.int32)
    out = _gather_scale(idx, embedding_table)
    return out.reshape(B, L, EMB)


# speedup vs baseline: 1.0840x; 1.0840x over previous
import functools

import jax
import jax.numpy as jnp
from jax import lax
from jax.experimental import pallas as pl
from jax.experimental.pallas import tpu as pltpu
from jax.experimental.pallas import tpu_sc as plsc

EMB = 64
SCALE = 8.0

NC = 2
NS = 16
NW = NC * NS
LANES = 16

B = 16384
L = 200
NB = 8
PER_W = B // NW
NIT = PER_W // NB


def _gather_scale(tok, table):
    mesh = plsc.VectorSubcoreMesh(core_axis_name="c", subcore_axis_name="s")

    @functools.partial(
        pl.kernel,
        mesh=mesh,
        out_type=jax.ShapeDtypeStruct((B, L, EMB), jnp.float32),
        scratch_types=[
            pltpu.VMEM((NB, L), jnp.int32),
            pltpu.VMEM((NB, L, EMB), jnp.float32),
            pltpu.SemaphoreType.DMA,
        ],
        compiler_params=pltpu.CompilerParams(use_tc_tiling_on_sc=False),
    )
    def k(tok_hbm, tab_hbm, out_hbm, idx_v, rows_v, sem):
        wid = lax.axis_index("s") * NC + lax.axis_index("c")
        base = wid * PER_W

        def body(i, carry):
            b0 = pl.multiple_of(base + i * NB, NB)
            pltpu.sync_copy(tok_hbm.at[pl.ds(b0, NB)], idx_v)
            for r in range(NB):
                pltpu.async_copy(
                    tab_hbm.at[idx_v.at[r, pl.ds(0, 128)]],
                    rows_v.at[r, pl.ds(0, 128)],
                    sem,
                )
                pltpu.async_copy(
                    tab_hbm.at[idx_v.at[r, pl.ds(128, 72)]],
                    rows_v.at[r, pl.ds(128, 72)],
                    sem,
                )
            for r in range(NB):
                pltpu.make_async_copy(
                    tab_hbm.at[idx_v.at[r, pl.ds(0, 128)]],
                    rows_v.at[r, pl.ds(0, 128)],
                    sem,
                ).wait()
                pltpu.make_async_copy(
                    tab_hbm.at[idx_v.at[r, pl.ds(128, 72)]],
                    rows_v.at[r, pl.ds(128, 72)],
                    sem,
                ).wait()

            def scale_l(l, c2):
                for r in range(NB):
                    for c in range(EMB // LANES):
                        sl = (r, l, pl.ds(c * LANES, LANES))
                        rows_v[sl] = rows_v[sl] * SCALE
                return c2

            lax.fori_loop(0, L, scale_l, 0)
            pltpu.sync_copy(rows_v, out_hbm.at[pl.ds(b0, NB)])
            return carry

        lax.fori_loop(0, NIT, body, 0)

    return k(tok, table)


def kernel(token_tensor, embedding_table):
    return _gather_scale(token_tensor.astype(jnp.int32), embedding_table)


# speedup vs baseline: 1.7006x; 1.5689x over previous
import functools

import jax
import jax.numpy as jnp
from jax import lax
from jax.experimental import pallas as pl
from jax.experimental.pallas import tpu as pltpu
from jax.experimental.pallas import tpu_sc as plsc

EMB = 64
SCALE = 8.0

NC = 2
NS = 16
NW = NC * NS
LANES = 16

B = 16384
L = 200
LPAD = 256
NB = 8
PER_W = B // NW
NIT = PER_W // NB


def _gather_scale(tok, table):
    mesh = plsc.VectorSubcoreMesh(core_axis_name="c", subcore_axis_name="s")

    @functools.partial(
        pl.kernel,
        mesh=mesh,
        out_type=jax.ShapeDtypeStruct((B, L, 128), jnp.float32),
        scratch_types=[
            pltpu.VMEM((NB, LPAD), jnp.int32),
            pltpu.VMEM((NB, L, EMB), jnp.float32),
            pltpu.SemaphoreType.DMA,
        ],
        compiler_params=pltpu.CompilerParams(use_tc_tiling_on_sc=False),
    )
    def k(tok_hbm, tab_hbm, out_hbm, idx_v, rows_v, sem):
        wid = lax.axis_index("s") * NC + lax.axis_index("c")
        base = wid * PER_W

        def body(i, carry):
            b0 = pl.multiple_of(base + i * NB, NB)
            pltpu.sync_copy(tok_hbm.at[pl.ds(b0, NB)], idx_v)
            for r in range(NB):
                pltpu.async_copy(
                    tab_hbm.at[idx_v.at[r, pl.ds(0, 128)]],
                    rows_v.at[r, pl.ds(0, 128)],
                    sem,
                )
                pltpu.async_copy(
                    tab_hbm.at[idx_v.at[r, pl.ds(128, 72)]],
                    rows_v.at[r, pl.ds(128, 72)],
                    sem,
                )
            for r in range(NB):
                pltpu.make_async_copy(
                    tab_hbm.at[idx_v.at[r, pl.ds(0, 128)]],
                    rows_v.at[r, pl.ds(0, 128)],
                    sem,
                ).wait()
                pltpu.make_async_copy(
                    tab_hbm.at[idx_v.at[r, pl.ds(128, 72)]],
                    rows_v.at[r, pl.ds(128, 72)],
                    sem,
                ).wait()

            def scale_l(l, c2):
                for r in range(NB):
                    for c in range(EMB // LANES):
                        sl = (r, l, pl.ds(c * LANES, LANES))
                        rows_v[sl] = rows_v[sl] * SCALE
                return c2

            lax.fori_loop(0, L, scale_l, 0)
            pltpu.sync_copy(
                rows_v, out_hbm.at[pl.ds(b0, NB), :, pl.ds(0, EMB)]
            )
            return carry

        lax.fori_loop(0, NIT, body, 0)

    return k(tok, table)


def kernel(token_tensor, embedding_table):
    tok = jnp.pad(token_tensor.astype(jnp.int32), ((0, 0), (0, LPAD - L)))
    out = _gather_scale(tok, embedding_table)
    return out[:, :, :EMB]
